# baseline (speedup 1.0000x reference)
import functools

import jax
import jax.numpy as jnp
from jax import lax
from jax.experimental import pallas as pl
from jax.experimental.pallas import tpu as pltpu
from jax.experimental.pallas import tpu_sc as plsc

_ROWS = 32768
_COLS = 2048
_NC = 2
_NS = 16
_NW = _NC * _NS
_RPW = _ROWS // _NW
_CH = 16
_NCHUNK = _RPW // _CH


@functools.partial(
    pl.kernel,
    mesh=plsc.VectorSubcoreMesh(core_axis_name="c", subcore_axis_name="s"),
    out_type=jax.ShapeDtypeStruct((_ROWS, _COLS), jnp.float32),
    scratch_types=[
        pltpu.VMEM((_CH, _COLS), jnp.float32),
        pltpu.VMEM((_CH, _COLS), jnp.float32),
        pltpu.SemaphoreType.DMA,
        pltpu.SemaphoreType.DMA,
        pltpu.SemaphoreType.DMA,
        pltpu.SemaphoreType.DMA,
    ],
)
def _sc_copy(x_hbm, o_hbm, buf0, buf1, si0, si1, so0, so1):
    wid = lax.axis_index("s") * _NC + lax.axis_index("c")
    base = wid * _RPW
    bufs = (buf0, buf1)
    isems = (si0, si1)
    osems = (so0, so1)

    in_copies = [None] * _NCHUNK
    out_copies = [None] * _NCHUNK
    for k in range(_NCHUNK):
        b = k & 1
        if k >= 2:
            out_copies[k - 2].wait()
        c_in = pltpu.make_async_copy(
            x_hbm.at[pl.ds(base + k * _CH, _CH)], bufs[b], isems[b]
        )
        c_in.start()
        in_copies[k] = c_in
        if k >= 1:
            in_copies[k - 1].wait()
            c_out = pltpu.make_async_copy(
                bufs[1 - b], o_hbm.at[pl.ds(base + (k - 1) * _CH, _CH)],
                osems[1 - b],
            )
            c_out.start()
            out_copies[k - 1] = c_out
    last = _NCHUNK - 1
    in_copies[last].wait()
    c_out = pltpu.make_async_copy(
        bufs[last & 1], o_hbm.at[pl.ds(base + last * _CH, _CH)],
        osems[last & 1],
    )
    c_out.start()
    out_copies[last] = c_out
    out_copies[last - 1].wait()
    out_copies[last].wait()


def kernel(x):
    b, s, d = x.shape
    out = _sc_copy(x.reshape(b * s, d))
    return out.reshape(b, s, d)


# speedup vs baseline: 1.2515x; 1.2515x over previous
import jax
import jax.numpy as jnp
from jax.experimental import pallas as pl
from jax.experimental.pallas import tpu as pltpu

_ROWS = 32768
_COLS = 2048
_CH = 512
_NCHUNK = _ROWS // _CH
_NBUF = 4


def _copy_body(x_ref, o_ref, ring, isems, osems):
    in_cp = [None] * _NCHUNK
    out_cp = [None] * _NCHUNK
    for k in range(_NCHUNK):
        r = k % _NBUF
        if k >= _NBUF:
            out_cp[k - _NBUF].wait()
        c_in = pltpu.make_async_copy(
            x_ref.at[pl.ds(k * _CH, _CH)], ring.at[r], isems.at[r]
        )
        c_in.start()
        in_cp[k] = c_in
        if k >= 1:
            in_cp[k - 1].wait()
            pr = (k - 1) % _NBUF
            c_out = pltpu.make_async_copy(
                ring.at[pr], o_ref.at[pl.ds((k - 1) * _CH, _CH)], osems.at[pr]
            )
            c_out.start()
            out_cp[k - 1] = c_out
    last = _NCHUNK - 1
    in_cp[last].wait()
    c_out = pltpu.make_async_copy(
        ring.at[last % _NBUF], o_ref.at[pl.ds(last * _CH, _CH)],
        osems.at[last % _NBUF],
    )
    c_out.start()
    out_cp[last] = c_out
    for k in range(_NCHUNK - _NBUF, _NCHUNK):
        out_cp[k].wait()


def kernel(x):
    b, s, d = x.shape
    xr = x.reshape(_ROWS, _COLS)
    out = pl.pallas_call(
        _copy_body,
        in_specs=[pl.BlockSpec(memory_space=pl.ANY)],
        out_specs=pl.BlockSpec(memory_space=pl.ANY),
        scratch_shapes=[
            pltpu.VMEM((_NBUF, _CH, _COLS), jnp.float32),
            pltpu.SemaphoreType.DMA((_NBUF,)),
            pltpu.SemaphoreType.DMA((_NBUF,)),
        ],
        out_shape=jax.ShapeDtypeStruct((_ROWS, _COLS), x.dtype),
    )(xr)
    return out.reshape(b, s, d)
